# baseline (speedup 1.0000x reference)
import functools

import jax
import jax.numpy as jnp
from jax import lax
from jax.experimental import pallas as pl
from jax.experimental.pallas import tpu as pltpu
from jax.experimental.pallas import tpu_sc as plsc

N_NODE, N_POD, N_SVC = 10000, 30000, 4000
T, F, H = 8, 64, 64
TOTAL = N_NODE + N_POD + N_SVC

NC, NS = 2, 16
KB = 128
KBC = 80
E_PAD_IN = 32768
E_PAD_SVC = 65536
EC_IN = 33280
EC_SVC = 66560
CNT_ROWS = 44032
CNT_PER_TILE = CNT_ROWS // NS
AGG_TAB = 30016
ZROWS = 1888
ZC_A = 344

_mesh = plsc.VectorSubcoreMesh(core_axis_name="c", subcore_axis_name="s",
                               num_cores=NC, num_subcores=NS)
_sc_params = pltpu.CompilerParams(use_tc_tiling_on_sc=False)


@functools.partial(
    pl.kernel,
    out_type=(jax.ShapeDtypeStruct((NC, CNT_ROWS, 16), jnp.float32),
              jax.ShapeDtypeStruct((NC, CNT_ROWS, 16), jnp.float32)),
    mesh=_mesh,
    scratch_types=[
        pltpu.VMEM_SHARED((CNT_ROWS, 16), jnp.float32),
        pltpu.VMEM_SHARED((CNT_ROWS, 16), jnp.float32),
        pltpu.VMEM((ZC_A, 16), jnp.float32),
        pltpu.VMEM((KB, 16), jnp.float32),
        pltpu.VMEM((KB,), jnp.int32),
    ],
    compiler_params=_sc_params,
)
def _sc_counts(sg_in, sg_ni, sg_svc, dg_in, dg_ni, dg_svc,
               cnt_src_out, cnt_dst_out, tab_s, tab_d, zbuf, onesbuf, idxbuf):
    c = lax.axis_index("c")
    s = lax.axis_index("s")
    wid = s * NC + c

    zero16 = jnp.zeros((16,), jnp.float32)
    e0 = jnp.where(lax.iota(jnp.int32, 16) == 0,
                   jnp.float32(1.0), jnp.float32(0.0))

    def fill_z(i, _):
        zbuf[i, :] = zero16
        return 0
    lax.fori_loop(0, ZC_A, fill_z, 0)

    def fill_o(i, _):
        onesbuf[i, :] = e0
        return 0
    lax.fori_loop(0, KB, fill_o, 0)

    r0 = s * CNT_PER_TILE

    def zero_tabs(i, _):
        pltpu.sync_copy(zbuf, tab_s.at[pl.ds(r0 + i * ZC_A, ZC_A)])
        pltpu.sync_copy(zbuf, tab_d.at[pl.ds(r0 + i * ZC_A, ZC_A)])
        return 0
    lax.fori_loop(0, CNT_PER_TILE // ZC_A, zero_tabs, 0)
    plsc.subcore_barrier()

    def scat(arr, tab, nb):
        base = wid * (nb * KB)

        def body(i, _):
            pltpu.sync_copy(arr.at[pl.ds(base + i * KB, KB)], idxbuf)
            pltpu.sync_copy(onesbuf, tab.at[idxbuf], add=True)
            return 0
        lax.fori_loop(0, nb, body, 0)

    scat(sg_in, tab_s, E_PAD_IN // (NC * NS * KB))
    scat(sg_ni, tab_s, E_PAD_IN // (NC * NS * KB))
    scat(sg_svc, tab_s, E_PAD_SVC // (NC * NS * KB))
    scat(dg_in, tab_d, E_PAD_IN // (NC * NS * KB))
    scat(dg_ni, tab_d, E_PAD_IN // (NC * NS * KB))
    scat(dg_svc, tab_d, E_PAD_SVC // (NC * NS * KB))
    plsc.subcore_barrier()

    pltpu.sync_copy(tab_s.at[pl.ds(r0, CNT_PER_TILE)],
                    cnt_src_out.at[c, pl.ds(r0, CNT_PER_TILE)])
    pltpu.sync_copy(tab_d.at[pl.ds(r0, CNT_PER_TILE)],
                    cnt_dst_out.at[c, pl.ds(r0, CNT_PER_TILE)])


@functools.partial(
    pl.kernel,
    out_type=jax.ShapeDtypeStruct((T, TOTAL, F), jnp.float32),
    mesh=_mesh,
    scratch_types=[
        pltpu.VMEM_SHARED((AGG_TAB, F), jnp.float32),
        pltpu.VMEM((KBC, F), jnp.float32),
        pltpu.VMEM((2, KBC), jnp.int32),
        pltpu.SemaphoreType.DMA,
        pltpu.VMEM((KBC, F), jnp.float32),
        pltpu.VMEM((2, KBC), jnp.int32),
        pltpu.SemaphoreType.DMA,
    ],
    compiler_params=_sc_params,
)
def _sc_agg(xs_flat, c_in, c_ni, c_svc, zhbm,
            agg_out, tab, rowA, idxA, semA, rowB, idxB, semB):
    c = lax.axis_index("c")
    s = lax.axis_index("s")
    bufs = ((rowA, idxA, semA), (rowB, idxB, semB))

    sections = (
        (c_in, N_NODE, 10240, EC_IN // (NS * KBC), 0, 1, 0, T),
        (c_ni, N_POD, AGG_TAB, EC_IN // (NS * KBC), N_NODE, 0, 0, 3),
        (c_ni, N_POD, AGG_TAB, EC_IN // (NS * KBC), N_NODE, 1, 3, T),
        (c_svc, N_SVC, 4096, EC_SVC // (NS * KBC), N_NODE + N_POD, 0, 0, T),
    )

    for (carr, n_dst, tabrows, nb, gbase, core, t_lo, t_hi) in sections:
        zpt = tabrows // NS
        cpt = n_dst // NS

        def start(buf, b, t, carr=carr):
            rb, ix, sm = buf
            pltpu.sync_copy(carr.at[b], ix)
            pltpu.async_copy(xs_flat.at[t].at[ix.at[0]], rb, sm)

        def finish(buf, t):
            rb, ix, sm = buf
            pltpu.make_async_copy(xs_flat.at[t].at[ix.at[0]], rb, sm).wait()
            pltpu.sync_copy(rb, tab.at[ix.at[1]], add=True)

        def drain(buf, t):
            rb, ix, sm = buf
            pltpu.make_async_copy(xs_flat.at[t].at[ix.at[0]], rb, sm).wait()

        def per_t(t, _, zpt=zpt, cpt=cpt, nb=nb, gbase=gbase,
                  start=start, finish=finish, drain=drain):
            bb = s * nb
            pltpu.sync_copy(zhbm.at[pl.ds(0, zpt)],
                            tab.at[pl.ds(s * zpt, zpt)])
            plsc.subcore_barrier()

            start(bufs[0], bb, t)

            def body(g, _):
                start(bufs[1], bb + 2 * g + 1, t)
                finish(bufs[0], t)
                start(bufs[0], bb + 2 * g + 2, t)
                finish(bufs[1], t)
                return 0
            lax.fori_loop(0, nb // 2, body, 0)
            drain(bufs[0], t)
            plsc.subcore_barrier()

            pltpu.sync_copy(tab.at[pl.ds(s * cpt, cpt)],
                            agg_out.at[t, pl.ds(gbase + s * cpt, cpt)])
            plsc.subcore_barrier()
            return 0

        @pl.when(c == core)
        def _run(per_t=per_t, t_lo=t_lo, t_hi=t_hi):
            lax.fori_loop(t_lo, t_hi, per_t, 0)


def _prescale_body(x_ref, cnt_ref, out_ref):
    deg = cnt_ref[0, :, 0] + cnt_ref[1, :, 0]
    nrm = lax.rsqrt(jnp.maximum(deg, 1.0))
    xsc = x_ref[...] * nrm[:, None]
    for t in range(T):
        out_ref[t] = xsc[:, t * F:(t + 1) * F]


def _prescale(xcat, cnt_src, R=400):
    n = xcat.shape[0]
    return pl.pallas_call(
        _prescale_body,
        grid=(n // R,),
        in_specs=[
            pl.BlockSpec((R, T * F), lambda i: (i, 0)),
            pl.BlockSpec((NC, R, 16), lambda i: (0, i, 0)),
        ],
        out_specs=pl.BlockSpec((T, R, F), lambda i: (0, i, 0)),
        out_shape=jax.ShapeDtypeStruct((T, n, F), jnp.float32),
    )(xcat, cnt_src)


def _conv_lstm_body(agg_ref, cnt_ref, W_ref, b_ref,
                    wc0_ref, bias0_ref, wc1_ref, bias1_ref,
                    out_ref):
    R = agg_ref.shape[1]
    deg = cnt_ref[0, :, 0] + cnt_ref[1, :, 0]
    nrm = lax.rsqrt(jnp.maximum(deg, 1.0))

    def lrelu(v):
        return jnp.where(v > 0, v, 0.01 * v)

    bf = jnp.bfloat16
    xs = []
    for t in range(T):
        y = jnp.dot((agg_ref[t] * nrm[:, None]).astype(bf),
                    W_ref[0, t].astype(bf),
                    preferred_element_type=jnp.float32) + b_ref[0, t]
        xs.append(lrelu(y))

    def lstm(x_list, wc, bias):
        wcb = wc.astype(bf)
        h = jnp.zeros((R, H), jnp.float32)
        cc = jnp.zeros((R, H), jnp.float32)
        outs = []
        for t in range(T):
            xh = jnp.concatenate([x_list[t], h], axis=1).astype(bf)
            g = jnp.dot(xh, wcb, preferred_element_type=jnp.float32) + bias
            i = jax.nn.sigmoid(g[:, 0 * H:1 * H])
            f = jax.nn.sigmoid(g[:, 1 * H:2 * H])
            gg = jnp.tanh(g[:, 2 * H:3 * H])
            o = jax.nn.sigmoid(g[:, 3 * H:4 * H])
            cc = f * cc + i * gg
            h = o * jnp.tanh(cc)
            outs.append(h)
        return outs

    h1 = lstm(xs, wc0_ref[...], bias0_ref[...])
    h2 = lstm(h1, wc1_ref[...], bias1_ref[...])
    out_ref[...] = jnp.stack(h2, axis=1)


def _conv_lstm(agg, cnt, Wall, ball, wc0, bias0, wc1, bias1, R):
    def typ(i):
        return ((i >= N_NODE // R).astype(jnp.int32)
                + (i >= (N_NODE + N_POD) // R).astype(jnp.int32))

    return pl.pallas_call(
        _conv_lstm_body,
        grid=(TOTAL // R,),
        in_specs=[
            pl.BlockSpec((T, R, F), lambda i: (0, i, 0)),
            pl.BlockSpec((NC, R, 16), lambda i: (0, i, 0)),
            pl.BlockSpec((1, T, F, H), lambda i: (typ(i), 0, 0, 0)),
            pl.BlockSpec((1, T, 1, H), lambda i: (typ(i), 0, 0, 0)),
            pl.BlockSpec((2 * H, 4 * H), lambda i: (0, 0)),
            pl.BlockSpec((1, 4 * H), lambda i: (0, 0)),
            pl.BlockSpec((2 * H, 4 * H), lambda i: (0, 0)),
            pl.BlockSpec((1, 4 * H), lambda i: (0, 0)),
        ],
        out_specs=pl.BlockSpec((R, T, H), lambda i: (i, 0, 0)),
        out_shape=jax.ShapeDtypeStruct((TOTAL, T, H), jnp.float32),
    )(agg, cnt, Wall, ball, wc0, bias0.reshape(1, 4 * H),
      wc1, bias1.reshape(1, 4 * H))


def _padto(a, n, fill):
    return jnp.concatenate(
        [a.astype(jnp.int32), jnp.full((n - a.shape[0],), fill, jnp.int32)])


def kernel(node_feat, pod_feat, svc_feat, svc_src, svc_dst, in_src, in_dst,
           ni_src, ni_dst, W_svc, b_svc, W_in, b_in, W_ni, b_ni,
           Wih0, Whh0, bih0, bhh0, Wih1, Whh1, bih1, bhh1):
    xcat = jnp.concatenate([node_feat.reshape(N_NODE, T * F),
                            pod_feat.reshape(N_POD, T * F),
                            svc_feat.reshape(N_SVC, T * F)], axis=0)

    sg_in = _padto(in_src + N_NODE, E_PAD_IN, TOTAL)
    sg_ni = _padto(ni_src, E_PAD_IN, TOTAL)
    sg_svc = _padto(svc_src + N_NODE + N_POD, E_PAD_SVC, TOTAL)
    dg_in = _padto(in_dst, E_PAD_IN, TOTAL)
    dg_ni = _padto(ni_dst + N_NODE, E_PAD_IN, TOTAL)
    dg_svc = _padto(svc_dst + N_NODE + N_POD, E_PAD_SVC, TOTAL)

    def _comb(src, dst, e_pad, n_dst):
        sA = _padto(src, e_pad, 0).reshape(-1, KBC)
        dA = _padto(dst, e_pad, n_dst).reshape(-1, KBC)
        comb = jnp.stack([sA, dA], axis=1)
        extra = jnp.stack([jnp.zeros((KBC,), jnp.int32),
                           jnp.full((KBC,), n_dst, jnp.int32)])[None]
        return jnp.concatenate([comb, extra], axis=0)

    c_in = _comb(in_src + N_NODE, in_dst, EC_IN, N_NODE)
    c_ni = _comb(ni_src, ni_dst, EC_IN, N_POD)
    c_svc = _comb(svc_src + N_NODE + N_POD, svc_dst, EC_SVC, N_SVC)
    zhbm = jnp.zeros((ZROWS, F), jnp.float32)

    cnt_src, cnt_dst = _sc_counts(sg_in, sg_ni, sg_svc, dg_in, dg_ni, dg_svc)
    xs = _prescale(xcat, cnt_src)
    agg = _sc_agg(xs, c_in, c_ni, c_svc, zhbm)

    bias0 = bih0 + bhh0
    bias1 = bih1 + bhh1
    wc0 = jnp.concatenate([Wih0.T, Whh0.T], axis=0)
    wc1 = jnp.concatenate([Wih1.T, Whh1.T], axis=0)
    Wall = jnp.stack([W_in, W_ni, W_svc])
    ball = jnp.stack([b_in, b_ni, b_svc]).reshape(3, T, 1, H)

    return _conv_lstm(agg, cnt_dst, Wall, ball,
                      wc0, bias0, wc1, bias1, 1000)


# speedup vs baseline: 1.0904x; 1.0904x over previous
import functools

import jax
import jax.numpy as jnp
from jax import lax
from jax.experimental import pallas as pl
from jax.experimental.pallas import tpu as pltpu
from jax.experimental.pallas import tpu_sc as plsc

N_NODE, N_POD, N_SVC = 10000, 30000, 4000
T, F, H = 8, 64, 64
TOTAL = N_NODE + N_POD + N_SVC

NC, NS = 2, 16
KB = 128
KBC = 64
E_PAD_IN = 32768
E_PAD_SVC = 65536
EC_IN = E_PAD_IN
EC_SVC = E_PAD_SVC
E_EXTRA = KBC
CNT_ROWS = 44032
CNT_PER_TILE = CNT_ROWS // NS
AGG_TAB = 30016
ZROWS = 1888
ZC_A = 344

_mesh = plsc.VectorSubcoreMesh(core_axis_name="c", subcore_axis_name="s",
                               num_cores=NC, num_subcores=NS)
_sc_params = pltpu.CompilerParams(use_tc_tiling_on_sc=False)


@functools.partial(
    pl.kernel,
    out_type=(jax.ShapeDtypeStruct((NC, CNT_ROWS, 16), jnp.float32),
              jax.ShapeDtypeStruct((NC, CNT_ROWS, 16), jnp.float32)),
    mesh=_mesh,
    scratch_types=[
        pltpu.VMEM_SHARED((CNT_ROWS, 16), jnp.float32),
        pltpu.VMEM_SHARED((CNT_ROWS, 16), jnp.float32),
        pltpu.VMEM((ZC_A, 16), jnp.float32),
        pltpu.VMEM((KB, 16), jnp.float32),
        pltpu.VMEM((KB,), jnp.int32),
    ],
    compiler_params=_sc_params,
)
def _sc_counts(sg_in, sg_ni, sg_svc, dg_in, dg_ni, dg_svc,
               cnt_src_out, cnt_dst_out, tab_s, tab_d, zbuf, onesbuf, idxbuf):
    c = lax.axis_index("c")
    s = lax.axis_index("s")
    wid = s * NC + c

    zero16 = jnp.zeros((16,), jnp.float32)
    e0 = jnp.where(lax.iota(jnp.int32, 16) == 0,
                   jnp.float32(1.0), jnp.float32(0.0))

    def fill_z(i, _):
        zbuf[i, :] = zero16
        return 0
    lax.fori_loop(0, ZC_A, fill_z, 0)

    def fill_o(i, _):
        onesbuf[i, :] = e0
        return 0
    lax.fori_loop(0, KB, fill_o, 0)

    r0 = s * CNT_PER_TILE

    def zero_tabs(i, _):
        pltpu.sync_copy(zbuf, tab_s.at[pl.ds(r0 + i * ZC_A, ZC_A)])
        pltpu.sync_copy(zbuf, tab_d.at[pl.ds(r0 + i * ZC_A, ZC_A)])
        return 0
    lax.fori_loop(0, CNT_PER_TILE // ZC_A, zero_tabs, 0)
    plsc.subcore_barrier()

    def scat(arr, tab, nb):
        base = wid * (nb * KB)

        def body(i, _):
            pltpu.sync_copy(arr.at[pl.ds(base + i * KB, KB)], idxbuf)
            pltpu.sync_copy(onesbuf, tab.at[idxbuf], add=True)
            return 0
        lax.fori_loop(0, nb, body, 0)

    scat(sg_in, tab_s, E_PAD_IN // (NC * NS * KB))
    scat(sg_ni, tab_s, E_PAD_IN // (NC * NS * KB))
    scat(sg_svc, tab_s, E_PAD_SVC // (NC * NS * KB))
    scat(dg_in, tab_d, E_PAD_IN // (NC * NS * KB))
    scat(dg_ni, tab_d, E_PAD_IN // (NC * NS * KB))
    scat(dg_svc, tab_d, E_PAD_SVC // (NC * NS * KB))
    plsc.subcore_barrier()

    pltpu.sync_copy(tab_s.at[pl.ds(r0, CNT_PER_TILE)],
                    cnt_src_out.at[c, pl.ds(r0, CNT_PER_TILE)])
    pltpu.sync_copy(tab_d.at[pl.ds(r0, CNT_PER_TILE)],
                    cnt_dst_out.at[c, pl.ds(r0, CNT_PER_TILE)])


@functools.partial(
    pl.kernel,
    out_type=jax.ShapeDtypeStruct((T, TOTAL, F), jnp.float32),
    mesh=_mesh,
    scratch_types=[
        pltpu.VMEM_SHARED((AGG_TAB, F), jnp.float32),
        pltpu.VMEM((KBC, F), jnp.float32),
        pltpu.VMEM((KBC,), jnp.int32),
        pltpu.VMEM((KBC,), jnp.int32),
        pltpu.SemaphoreType.DMA,
        pltpu.VMEM((KBC, F), jnp.float32),
        pltpu.VMEM((KBC,), jnp.int32),
        pltpu.VMEM((KBC,), jnp.int32),
        pltpu.SemaphoreType.DMA,
    ],
    compiler_params=_sc_params,
)
def _sc_agg(xs_flat, s8_in, s8_ni, s8_svc, d_in, d_ni, d_svc, zhbm,
            agg_out, tab, rowA, idxsA, idxdA, semA, rowB, idxsB, idxdB, semB):
    c = lax.axis_index("c")
    s = lax.axis_index("s")
    bufs = ((rowA, idxsA, idxdA, semA), (rowB, idxsB, idxdB, semB))

    sections = (
        (s8_in, d_in, N_NODE, 10240, EC_IN // (NS * KBC), 0, 1, 0, T),
        (s8_ni, d_ni, N_POD, AGG_TAB, EC_IN // (NS * KBC), N_NODE, 0, 0, 3),
        (s8_ni, d_ni, N_POD, AGG_TAB, EC_IN // (NS * KBC), N_NODE, 1, 3, T),
        (s8_svc, d_svc, N_SVC, 4096, EC_SVC // (NS * KBC),
         N_NODE + N_POD, 0, 0, T),
    )

    for (srcarr, dstarr, n_dst, tabrows, nb, gbase, core,
         t_lo, t_hi) in sections:
        zpt = tabrows // NS
        cpt = n_dst // NS
        ebase = s * (nb * KBC)

        def start(buf, off, t, srcarr=srcarr, dstarr=dstarr):
            rb, ixs, ixd, sm = buf
            pltpu.sync_copy(srcarr.at[pl.ds(off, KBC)], ixs)
            pltpu.async_copy(xs_flat.at[t].at[ixs], rb, sm)
            pltpu.sync_copy(dstarr.at[pl.ds(off, KBC)], ixd)

        def finish(buf, t):
            rb, ixs, ixd, sm = buf
            pltpu.make_async_copy(xs_flat.at[t].at[ixs], rb, sm).wait()
            pltpu.sync_copy(rb, tab.at[ixd], add=True)

        def drain(buf, t):
            rb, ixs, ixd, sm = buf
            pltpu.make_async_copy(xs_flat.at[t].at[ixs], rb, sm).wait()

        def per_t(t, _, zpt=zpt, cpt=cpt, nb=nb, gbase=gbase, ebase=ebase,
                  start=start, finish=finish, drain=drain):
            pltpu.sync_copy(zhbm.at[pl.ds(0, zpt)],
                            tab.at[pl.ds(s * zpt, zpt)])
            plsc.subcore_barrier()

            start(bufs[0], ebase, t)

            def body(g, _):
                start(bufs[1], ebase + (2 * g + 1) * KBC, t)
                finish(bufs[0], t)
                start(bufs[0], ebase + (2 * g + 2) * KBC, t)
                finish(bufs[1], t)
                return 0
            lax.fori_loop(0, nb // 2, body, 0)
            drain(bufs[0], t)
            plsc.subcore_barrier()

            pltpu.sync_copy(tab.at[pl.ds(s * cpt, cpt)],
                            agg_out.at[t, pl.ds(gbase + s * cpt, cpt)])
            plsc.subcore_barrier()
            return 0

        @pl.when(c == core)
        def _run(per_t=per_t, t_lo=t_lo, t_hi=t_hi):
            lax.fori_loop(t_lo, t_hi, per_t, 0)


def _prescale_body(x_ref, cnt_ref, out_ref):
    deg = cnt_ref[0, :, 0] + cnt_ref[1, :, 0]
    nrm = lax.rsqrt(jnp.maximum(deg, 1.0))
    xsc = x_ref[...] * nrm[:, None]
    for t in range(T):
        out_ref[t] = xsc[:, t * F:(t + 1) * F]


def _prescale(xcat, cnt_src, R=400):
    n = xcat.shape[0]
    return pl.pallas_call(
        _prescale_body,
        grid=(n // R,),
        in_specs=[
            pl.BlockSpec((R, T * F), lambda i: (i, 0)),
            pl.BlockSpec((NC, R, 16), lambda i: (0, i, 0)),
        ],
        out_specs=pl.BlockSpec((T, R, F), lambda i: (0, i, 0)),
        out_shape=jax.ShapeDtypeStruct((T, n, F), jnp.float32),
    )(xcat, cnt_src)


def _conv_lstm_body(agg_ref, cnt_ref, W_ref, b_ref,
                    wc0_ref, bias0_ref, wc1_ref, bias1_ref,
                    out_ref):
    R = agg_ref.shape[1]
    deg = cnt_ref[0, :, 0] + cnt_ref[1, :, 0]
    nrm = lax.rsqrt(jnp.maximum(deg, 1.0))

    def lrelu(v):
        return jnp.where(v > 0, v, 0.01 * v)

    bf = jnp.bfloat16
    xs = []
    for t in range(T):
        y = jnp.dot((agg_ref[t] * nrm[:, None]).astype(bf),
                    W_ref[0, t].astype(bf),
                    preferred_element_type=jnp.float32) + b_ref[0, t]
        xs.append(lrelu(y))

    def lstm(x_list, wc, bias):
        wcb = wc.astype(bf)
        h = jnp.zeros((R, H), jnp.float32)
        cc = jnp.zeros((R, H), jnp.float32)
        outs = []
        for t in range(T):
            xh = jnp.concatenate([x_list[t], h], axis=1).astype(bf)
            g = jnp.dot(xh, wcb, preferred_element_type=jnp.float32) + bias
            i = jax.nn.sigmoid(g[:, 0 * H:1 * H])
            f = jax.nn.sigmoid(g[:, 1 * H:2 * H])
            gg = jnp.tanh(g[:, 2 * H:3 * H])
            o = jax.nn.sigmoid(g[:, 3 * H:4 * H])
            cc = f * cc + i * gg
            h = o * jnp.tanh(cc)
            outs.append(h)
        return outs

    h1 = lstm(xs, wc0_ref[...], bias0_ref[...])
    h2 = lstm(h1, wc1_ref[...], bias1_ref[...])
    out_ref[...] = jnp.stack(h2, axis=1)


def _conv_lstm(agg, cnt, Wall, ball, wc0, bias0, wc1, bias1, R):
    def typ(i):
        return ((i >= N_NODE // R).astype(jnp.int32)
                + (i >= (N_NODE + N_POD) // R).astype(jnp.int32))

    return pl.pallas_call(
        _conv_lstm_body,
        grid=(TOTAL // R,),
        in_specs=[
            pl.BlockSpec((T, R, F), lambda i: (0, i, 0)),
            pl.BlockSpec((NC, R, 16), lambda i: (0, i, 0)),
            pl.BlockSpec((1, T, F, H), lambda i: (typ(i), 0, 0, 0)),
            pl.BlockSpec((1, T, 1, H), lambda i: (typ(i), 0, 0, 0)),
            pl.BlockSpec((2 * H, 4 * H), lambda i: (0, 0)),
            pl.BlockSpec((1, 4 * H), lambda i: (0, 0)),
            pl.BlockSpec((2 * H, 4 * H), lambda i: (0, 0)),
            pl.BlockSpec((1, 4 * H), lambda i: (0, 0)),
        ],
        out_specs=pl.BlockSpec((R, T, H), lambda i: (i, 0, 0)),
        out_shape=jax.ShapeDtypeStruct((TOTAL, T, H), jnp.float32),
    )(agg, cnt, Wall, ball, wc0, bias0.reshape(1, 4 * H),
      wc1, bias1.reshape(1, 4 * H))


def _padto(a, n, fill):
    return jnp.concatenate(
        [a.astype(jnp.int32), jnp.full((n - a.shape[0],), fill, jnp.int32)])


def kernel(node_feat, pod_feat, svc_feat, svc_src, svc_dst, in_src, in_dst,
           ni_src, ni_dst, W_svc, b_svc, W_in, b_in, W_ni, b_ni,
           Wih0, Whh0, bih0, bhh0, Wih1, Whh1, bih1, bhh1):
    xcat = jnp.concatenate([node_feat.reshape(N_NODE, T * F),
                            pod_feat.reshape(N_POD, T * F),
                            svc_feat.reshape(N_SVC, T * F)], axis=0)

    sg_in = _padto(in_src + N_NODE, E_PAD_IN, TOTAL)
    sg_ni = _padto(ni_src, E_PAD_IN, TOTAL)
    sg_svc = _padto(svc_src + N_NODE + N_POD, E_PAD_SVC, TOTAL)
    dg_in = _padto(in_dst, E_PAD_IN, TOTAL)
    dg_ni = _padto(ni_dst + N_NODE, E_PAD_IN, TOTAL)
    dg_svc = _padto(svc_dst + N_NODE + N_POD, E_PAD_SVC, TOTAL)

    s8_in = _padto(in_src + N_NODE, EC_IN + E_EXTRA, 0)
    s8_ni = _padto(ni_src, EC_IN + E_EXTRA, 0)
    s8_svc = _padto(svc_src + N_NODE + N_POD, EC_SVC + E_EXTRA, 0)
    d_in = _padto(in_dst, EC_IN + E_EXTRA, N_NODE)
    d_ni = _padto(ni_dst, EC_IN + E_EXTRA, N_POD)
    d_svc = _padto(svc_dst, EC_SVC + E_EXTRA, N_SVC)
    zhbm = jnp.zeros((ZROWS, F), jnp.float32)

    cnt_src, cnt_dst = _sc_counts(sg_in, sg_ni, sg_svc, dg_in, dg_ni, dg_svc)
    xs = _prescale(xcat, cnt_src)
    agg = _sc_agg(xs, s8_in, s8_ni, s8_svc, d_in, d_ni, d_svc, zhbm)

    bias0 = bih0 + bhh0
    bias1 = bih1 + bhh1
    wc0 = jnp.concatenate([Wih0.T, Whh0.T], axis=0)
    wc1 = jnp.concatenate([Wih1.T, Whh1.T], axis=0)
    Wall = jnp.stack([W_in, W_ni, W_svc])
    ball = jnp.stack([b_in, b_ni, b_svc]).reshape(3, T, 1, H)

    return _conv_lstm(agg, cnt_dst, Wall, ball,
                      wc0, bias0, wc1, bias1, 1000)


# speedup vs baseline: 1.1024x; 1.0110x over previous
import functools

import jax
import jax.numpy as jnp
from jax import lax
from jax.experimental import pallas as pl
from jax.experimental.pallas import tpu as pltpu
from jax.experimental.pallas import tpu_sc as plsc

N_NODE, N_POD, N_SVC = 10000, 30000, 4000
T, F, H = 8, 64, 64
TOTAL = N_NODE + N_POD + N_SVC

NC, NS = 2, 16
KB = 128
KBC = 64
E_PAD_IN = 32768
E_PAD_SVC = 65536
EC_IN = E_PAD_IN
EC_SVC = E_PAD_SVC
E_EXTRA = KBC
CNT_ROWS = 44032
CNT_PER_TILE = CNT_ROWS // NS
AGG_TAB = 30016
ZROWS = 1888
ZC_A = 344

_mesh = plsc.VectorSubcoreMesh(core_axis_name="c", subcore_axis_name="s",
                               num_cores=NC, num_subcores=NS)
_sc_params = pltpu.CompilerParams(use_tc_tiling_on_sc=False)


@functools.partial(
    pl.kernel,
    out_type=(jax.ShapeDtypeStruct((NC, CNT_ROWS, 16), jnp.float32),
              jax.ShapeDtypeStruct((NC, CNT_ROWS, 16), jnp.float32)),
    mesh=_mesh,
    scratch_types=[
        pltpu.VMEM_SHARED((CNT_ROWS, 16), jnp.float32),
        pltpu.VMEM_SHARED((CNT_ROWS, 16), jnp.float32),
        pltpu.VMEM((ZC_A, 16), jnp.float32),
        pltpu.VMEM((KB, 16), jnp.float32),
        pltpu.VMEM((KB,), jnp.int32),
    ],
    compiler_params=_sc_params,
)
def _sc_counts(sg_in, sg_ni, sg_svc, dg_in, dg_ni, dg_svc,
               cnt_src_out, cnt_dst_out, tab_s, tab_d, zbuf, onesbuf, idxbuf):
    c = lax.axis_index("c")
    s = lax.axis_index("s")
    wid = s * NC + c

    zero16 = jnp.zeros((16,), jnp.float32)
    e0 = jnp.where(lax.iota(jnp.int32, 16) == 0,
                   jnp.float32(1.0), jnp.float32(0.0))

    def fill_z(i, _):
        zbuf[i, :] = zero16
        return 0
    lax.fori_loop(0, ZC_A, fill_z, 0)

    def fill_o(i, _):
        onesbuf[i, :] = e0
        return 0
    lax.fori_loop(0, KB, fill_o, 0)

    r0 = s * CNT_PER_TILE

    def zero_tabs(i, _):
        pltpu.sync_copy(zbuf, tab_s.at[pl.ds(r0 + i * ZC_A, ZC_A)])
        pltpu.sync_copy(zbuf, tab_d.at[pl.ds(r0 + i * ZC_A, ZC_A)])
        return 0
    lax.fori_loop(0, CNT_PER_TILE // ZC_A, zero_tabs, 0)
    plsc.subcore_barrier()

    def scat(arr, tab, nb):
        base = wid * (nb * KB)

        def body(i, _):
            pltpu.sync_copy(arr.at[pl.ds(base + i * KB, KB)], idxbuf)
            pltpu.sync_copy(onesbuf, tab.at[idxbuf], add=True)
            return 0
        lax.fori_loop(0, nb, body, 0)

    scat(sg_in, tab_s, E_PAD_IN // (NC * NS * KB))
    scat(sg_ni, tab_s, E_PAD_IN // (NC * NS * KB))
    scat(sg_svc, tab_s, E_PAD_SVC // (NC * NS * KB))
    scat(dg_in, tab_d, E_PAD_IN // (NC * NS * KB))
    scat(dg_ni, tab_d, E_PAD_IN // (NC * NS * KB))
    scat(dg_svc, tab_d, E_PAD_SVC // (NC * NS * KB))
    plsc.subcore_barrier()

    pltpu.sync_copy(tab_s.at[pl.ds(r0, CNT_PER_TILE)],
                    cnt_src_out.at[c, pl.ds(r0, CNT_PER_TILE)])
    pltpu.sync_copy(tab_d.at[pl.ds(r0, CNT_PER_TILE)],
                    cnt_dst_out.at[c, pl.ds(r0, CNT_PER_TILE)])


@functools.partial(
    pl.kernel,
    out_type=jax.ShapeDtypeStruct((T, TOTAL, F), jnp.float32),
    mesh=_mesh,
    scratch_types=[
        pltpu.VMEM_SHARED((AGG_TAB, F), jnp.float32),
        pltpu.VMEM((KBC, F), jnp.float32),
        pltpu.VMEM((KBC,), jnp.int32),
        pltpu.VMEM((KBC,), jnp.int32),
        pltpu.SemaphoreType.DMA,
        pltpu.VMEM((KBC, F), jnp.float32),
        pltpu.VMEM((KBC,), jnp.int32),
        pltpu.VMEM((KBC,), jnp.int32),
        pltpu.SemaphoreType.DMA,
    ],
    compiler_params=_sc_params,
)
def _sc_agg(xs_flat, s8_in, s8_ni, s8_svc, d_in, d_ni, d_svc, zhbm,
            agg_out, tab, rowA, idxsA, idxdA, semA, rowB, idxsB, idxdB, semB):
    c = lax.axis_index("c")
    s = lax.axis_index("s")
    bufs = ((rowA, idxsA, idxdA, semA), (rowB, idxsB, idxdB, semB))

    sections = (
        (s8_in, d_in, N_NODE, 10240, EC_IN // (NS * KBC), 0, 1, 0, T),
        (s8_ni, d_ni, N_POD, AGG_TAB, EC_IN // (NS * KBC), N_NODE, 0, 0, 3),
        (s8_ni, d_ni, N_POD, AGG_TAB, EC_IN // (NS * KBC), N_NODE, 1, 3, T),
        (s8_svc, d_svc, N_SVC, 4096, EC_SVC // (NS * KBC),
         N_NODE + N_POD, 0, 0, T),
    )

    for (srcarr, dstarr, n_dst, tabrows, nb, gbase, core,
         t_lo, t_hi) in sections:
        zpt = tabrows // NS
        cpt = n_dst // NS
        ebase = s * (nb * KBC)

        def start(buf, off, t, srcarr=srcarr, dstarr=dstarr):
            rb, ixs, ixd, sm = buf
            pltpu.sync_copy(srcarr.at[pl.ds(off, KBC)], ixs)
            pltpu.async_copy(xs_flat.at[t].at[ixs], rb, sm)
            pltpu.sync_copy(dstarr.at[pl.ds(off, KBC)], ixd)

        def finish(buf, t):
            rb, ixs, ixd, sm = buf
            pltpu.make_async_copy(xs_flat.at[t].at[ixs], rb, sm).wait()
            pltpu.sync_copy(rb, tab.at[ixd], add=True)

        def drain(buf, t):
            rb, ixs, ixd, sm = buf
            pltpu.make_async_copy(xs_flat.at[t].at[ixs], rb, sm).wait()

        def per_t(t, _, zpt=zpt, cpt=cpt, nb=nb, gbase=gbase, ebase=ebase,
                  start=start, finish=finish, drain=drain):
            pltpu.sync_copy(zhbm.at[pl.ds(0, zpt)],
                            tab.at[pl.ds(s * zpt, zpt)])
            plsc.subcore_barrier()

            start(bufs[0], ebase, t)

            def body(g, _):
                start(bufs[1], ebase + (2 * g + 1) * KBC, t)
                finish(bufs[0], t)
                start(bufs[0], ebase + (2 * g + 2) * KBC, t)
                finish(bufs[1], t)
                return 0
            lax.fori_loop(0, nb // 2, body, 0)
            drain(bufs[0], t)
            plsc.subcore_barrier()

            pltpu.sync_copy(tab.at[pl.ds(s * cpt, cpt)],
                            agg_out.at[t, pl.ds(gbase + s * cpt, cpt)])
            plsc.subcore_barrier()
            return 0

        @pl.when(c == core)
        def _run(per_t=per_t, t_lo=t_lo, t_hi=t_hi):
            lax.fori_loop(t_lo, t_hi, per_t, 0)


def _prescale_body(x_ref, cnt_ref, out_ref):
    deg = cnt_ref[0, :, 0] + cnt_ref[1, :, 0]
    nrm = lax.rsqrt(jnp.maximum(deg, 1.0))
    xsc = x_ref[...] * nrm[:, None]
    for t in range(T):
        out_ref[t] = xsc[:, t * F:(t + 1) * F]


def _prescale(xcat, cnt_src, R=1000):
    n = xcat.shape[0]
    return pl.pallas_call(
        _prescale_body,
        grid=(n // R,),
        in_specs=[
            pl.BlockSpec((R, T * F), lambda i: (i, 0)),
            pl.BlockSpec((NC, R, 16), lambda i: (0, i, 0)),
        ],
        out_specs=pl.BlockSpec((T, R, F), lambda i: (0, i, 0)),
        out_shape=jax.ShapeDtypeStruct((T, n, F), jnp.float32),
    )(xcat, cnt_src)


def _conv_lstm_body(agg_ref, cnt_ref, W_ref, b_ref,
                    wc0_ref, bias0_ref, wc1_ref, bias1_ref,
                    out_ref):
    R = agg_ref.shape[1]
    deg = cnt_ref[0, :, 0] + cnt_ref[1, :, 0]
    nrm = lax.rsqrt(jnp.maximum(deg, 1.0))

    def lrelu(v):
        return jnp.where(v > 0, v, 0.01 * v)

    bf = jnp.bfloat16
    xs = []
    for t in range(T):
        y = jnp.dot((agg_ref[t] * nrm[:, None]).astype(bf),
                    W_ref[0, t].astype(bf),
                    preferred_element_type=jnp.float32) + b_ref[0, t]
        xs.append(lrelu(y))

    def lstm(x_list, wc, bias):
        wcb = wc.astype(bf)
        h = jnp.zeros((R, H), jnp.float32)
        cc = jnp.zeros((R, H), jnp.float32)
        outs = []
        for t in range(T):
            xh = jnp.concatenate([x_list[t], h], axis=1).astype(bf)
            g = jnp.dot(xh, wcb, preferred_element_type=jnp.float32) + bias
            i = jax.nn.sigmoid(g[:, 0 * H:1 * H])
            f = jax.nn.sigmoid(g[:, 1 * H:2 * H])
            gg = jnp.tanh(g[:, 2 * H:3 * H])
            o = jax.nn.sigmoid(g[:, 3 * H:4 * H])
            cc = f * cc + i * gg
            h = o * jnp.tanh(cc)
            outs.append(h)
        return outs

    h1 = lstm(xs, wc0_ref[...], bias0_ref[...])
    h2 = lstm(h1, wc1_ref[...], bias1_ref[...])
    out_ref[...] = jnp.stack(h2, axis=1)


def _conv_lstm(agg, cnt, Wall, ball, wc0, bias0, wc1, bias1, R):
    def typ(i):
        return ((i >= N_NODE // R).astype(jnp.int32)
                + (i >= (N_NODE + N_POD) // R).astype(jnp.int32))

    return pl.pallas_call(
        _conv_lstm_body,
        grid=(TOTAL // R,),
        in_specs=[
            pl.BlockSpec((T, R, F), lambda i: (0, i, 0)),
            pl.BlockSpec((NC, R, 16), lambda i: (0, i, 0)),
            pl.BlockSpec((1, T, F, H), lambda i: (typ(i), 0, 0, 0)),
            pl.BlockSpec((1, T, 1, H), lambda i: (typ(i), 0, 0, 0)),
            pl.BlockSpec((2 * H, 4 * H), lambda i: (0, 0)),
            pl.BlockSpec((1, 4 * H), lambda i: (0, 0)),
            pl.BlockSpec((2 * H, 4 * H), lambda i: (0, 0)),
            pl.BlockSpec((1, 4 * H), lambda i: (0, 0)),
        ],
        out_specs=pl.BlockSpec((R, T, H), lambda i: (i, 0, 0)),
        out_shape=jax.ShapeDtypeStruct((TOTAL, T, H), jnp.float32),
    )(agg, cnt, Wall, ball, wc0, bias0.reshape(1, 4 * H),
      wc1, bias1.reshape(1, 4 * H))


def _padto(a, n, fill):
    return jnp.concatenate(
        [a.astype(jnp.int32), jnp.full((n - a.shape[0],), fill, jnp.int32)])


def kernel(node_feat, pod_feat, svc_feat, svc_src, svc_dst, in_src, in_dst,
           ni_src, ni_dst, W_svc, b_svc, W_in, b_in, W_ni, b_ni,
           Wih0, Whh0, bih0, bhh0, Wih1, Whh1, bih1, bhh1):
    xcat = jnp.concatenate([node_feat.reshape(N_NODE, T * F),
                            pod_feat.reshape(N_POD, T * F),
                            svc_feat.reshape(N_SVC, T * F)], axis=0)

    sg_in = _padto(in_src + N_NODE, E_PAD_IN, TOTAL)
    sg_ni = _padto(ni_src, E_PAD_IN, TOTAL)
    sg_svc = _padto(svc_src + N_NODE + N_POD, E_PAD_SVC, TOTAL)
    dg_in = _padto(in_dst, E_PAD_IN, TOTAL)
    dg_ni = _padto(ni_dst + N_NODE, E_PAD_IN, TOTAL)
    dg_svc = _padto(svc_dst + N_NODE + N_POD, E_PAD_SVC, TOTAL)

    s8_in = _padto(in_src + N_NODE, EC_IN + E_EXTRA, 0)
    s8_ni = _padto(ni_src, EC_IN + E_EXTRA, 0)
    s8_svc = _padto(svc_src + N_NODE + N_POD, EC_SVC + E_EXTRA, 0)
    d_in = _padto(in_dst, EC_IN + E_EXTRA, N_NODE)
    d_ni = _padto(ni_dst, EC_IN + E_EXTRA, N_POD)
    d_svc = _padto(svc_dst, EC_SVC + E_EXTRA, N_SVC)
    zhbm = jnp.zeros((ZROWS, F), jnp.float32)

    cnt_src, cnt_dst = _sc_counts(sg_in, sg_ni, sg_svc, dg_in, dg_ni, dg_svc)
    xs = _prescale(xcat, cnt_src)
    agg = _sc_agg(xs, s8_in, s8_ni, s8_svc, d_in, d_ni, d_svc, zhbm)

    bias0 = bih0 + bhh0
    bias1 = bih1 + bhh1
    wc0 = jnp.concatenate([Wih0.T, Whh0.T], axis=0)
    wc1 = jnp.concatenate([Wih1.T, Whh1.T], axis=0)
    Wall = jnp.stack([W_in, W_ni, W_svc])
    ball = jnp.stack([b_in, b_ni, b_svc]).reshape(3, T, 1, H)

    return _conv_lstm(agg, cnt_dst, Wall, ball,
                      wc0, bias0, wc1, bias1, 2000)
